# Initial kernel scaffold; baseline (speedup 1.0000x reference)
import jax
import jax.numpy as jnp
from jax.experimental import pallas as pl


def kernel(x, tables):
    raise NotImplementedError("write your pallas kernel here")


# speedup vs baseline: 1.1487x; 1.1487x over previous
import functools

import jax
import jax.numpy as jnp
from jax import lax
from jax.experimental import pallas as pl
from jax.experimental.pallas import tpu as pltpu
from jax.experimental.pallas import tpu_sc as plsc

_NUM_FIELDS = 26
_VOCAB = 100000
_EMBED_DIM = 16
_BATCH = 16384

_NC = 2
_NS = 16
_NW = _NC * _NS

_ROWS_PER_WORKER = _BATCH // _NW
_CHUNK_ROWS = 128
_CHUNK = _CHUNK_ROWS * _NUM_FIELDS
_NCHUNKS = _ROWS_PER_WORKER // _CHUNK_ROWS
_LANES = 16


def _body(x_hbm, off_hbm, tbl_hbm, out_hbm, raw_v, idx_v, off_v, rows_v, sem):
    wid = lax.axis_index("s") * _NC + lax.axis_index("c")
    pltpu.sync_copy(off_hbm, off_v)

    def chunk(n, _):
        base = (wid * _NCHUNKS + n) * _CHUNK
        pltpu.sync_copy(x_hbm.at[pl.ds(base, _CHUNK)], raw_v)

        def add(j, _):
            s = pl.ds(j * _LANES, _LANES)
            idx_v[s] = raw_v[s] + off_v[s]
            return 0

        lax.fori_loop(0, _CHUNK // _LANES, add, 0)
        pltpu.async_copy(tbl_hbm.at[idx_v], rows_v, sem).wait()
        pltpu.sync_copy(rows_v, out_hbm.at[pl.ds(base, _CHUNK)])
        return 0

    lax.fori_loop(0, _NCHUNKS, chunk, 0)


@jax.jit
def kernel(x, tables):
    x_flat = x.astype(jnp.int32).reshape(-1)
    tbl_flat = tables.reshape(_NUM_FIELDS * _VOCAB, _EMBED_DIM)
    offsets = jnp.tile(
        jnp.arange(_NUM_FIELDS, dtype=jnp.int32) * _VOCAB, _CHUNK_ROWS
    )

    mesh = plsc.VectorSubcoreMesh(core_axis_name="c", subcore_axis_name="s")
    gather = pl.kernel(
        _body,
        out_type=jax.ShapeDtypeStruct(
            (_BATCH * _NUM_FIELDS, _EMBED_DIM), jnp.float32
        ),
        mesh=mesh,
        scratch_types=[
            pltpu.VMEM((_CHUNK,), jnp.int32),
            pltpu.VMEM((_CHUNK,), jnp.int32),
            pltpu.VMEM((_CHUNK,), jnp.int32),
            pltpu.VMEM((_CHUNK, _EMBED_DIM), jnp.float32),
            pltpu.SemaphoreType.DMA,
        ],
        compiler_params=pltpu.CompilerParams(use_tc_tiling_on_sc=False),
    )
    out = gather(x_flat, offsets, tbl_flat)
    return out.reshape(_BATCH, _NUM_FIELDS * _EMBED_DIM)
